# baseline (speedup 1.0000x reference)
import jax
import jax.numpy as jnp
from jax.experimental import pallas as pl
from jax.experimental.pallas import tpu as pltpu

_LAMBDA = 0.1
_B = 1024
_C = 100000

_BC = 5000
_GRID = _C // _BC


def _body(x_ref, t_ref, o_ref, tacc_ref, jacc_ref):
    @pl.when(pl.program_id(0) == 0)
    def _init():
        tacc_ref[...] = jnp.zeros((8, _B), jnp.float32)
        jacc_ref[...] = jnp.zeros((8, _B), jnp.float32)

    x = x_ref[...]
    t = t_ref[0]
    cls = pl.program_id(0) * _BC + jax.lax.broadcasted_iota(jnp.int32, (_BC, _B), 0)
    sel = jnp.where(cls == t, x, 0.0)
    tacc_ref[...] += jnp.sum(x.reshape(_BC // 8, 8, _B), axis=0)
    jacc_ref[...] += jnp.sum(sel.reshape(_BC // 8, 8, _B), axis=0)

    @pl.when(pl.program_id(0) == _GRID - 1)
    def _fin():
        total = jnp.sum(tacc_ref[...])
        picked = jnp.sum(jacc_ref[...])
        o_ref[0, 0] = (picked - total / _C) * (_LAMBDA / _B)


def kernel(output, target):
    xt = output.T
    tgt3d = target.astype(jnp.int32).reshape(1, 1, _B)
    out = pl.pallas_call(
        _body,
        grid=(_GRID,),
        in_specs=[
            pl.BlockSpec((_BC, _B), lambda i: (i, 0)),
            pl.BlockSpec((1, 1, _B), lambda i: (0, 0, 0)),
        ],
        out_specs=pl.BlockSpec((1, 1), lambda i: (0, 0), memory_space=pltpu.SMEM),
        out_shape=jax.ShapeDtypeStruct((1, 1), jnp.float32),
        scratch_shapes=[
            pltpu.VMEM((8, _B), jnp.float32),
            pltpu.VMEM((8, _B), jnp.float32),
        ],
    )(xt, tgt3d)
    return out.reshape(())


# speedup vs baseline: 1.0160x; 1.0160x over previous
import jax
import jax.numpy as jnp
from jax.experimental import pallas as pl
from jax.experimental.pallas import tpu as pltpu

_LAMBDA = 0.1
_B = 1024
_C = 100000

_BC = 4000
_GRID = _C // _BC

_W1 = 1.0 - 1.0 / _C
_W0 = -1.0 / _C


def _body(x_ref, t_ref, o_ref, acc_ref):
    @pl.when(pl.program_id(0) == 0)
    def _init():
        acc_ref[...] = jnp.zeros((8, _B), jnp.float32)

    x = x_ref[...]
    t = t_ref[0]
    cls = pl.program_id(0) * _BC + jax.lax.broadcasted_iota(jnp.int32, (_BC, _B), 0)
    w = jnp.where(cls == t, jnp.float32(_W1), jnp.float32(_W0))
    acc_ref[...] += jnp.sum((w * x).reshape(_BC // 8, 8, _B), axis=0)

    @pl.when(pl.program_id(0) == _GRID - 1)
    def _fin():
        o_ref[0, 0] = jnp.sum(acc_ref[...]) * (_LAMBDA / _B)


def kernel(output, target):
    xt = output.T
    tgt3d = target.astype(jnp.int32).reshape(1, 1, _B)
    out = pl.pallas_call(
        _body,
        grid=(_GRID,),
        in_specs=[
            pl.BlockSpec((_BC, _B), lambda i: (i, 0)),
            pl.BlockSpec((1, 1, _B), lambda i: (0, 0, 0)),
        ],
        out_specs=pl.BlockSpec((1, 1), lambda i: (0, 0), memory_space=pltpu.SMEM),
        out_shape=jax.ShapeDtypeStruct((1, 1), jnp.float32),
        scratch_shapes=[
            pltpu.VMEM((8, _B), jnp.float32),
        ],
    )(xt, tgt3d)
    return out.reshape(())
